# Initial kernel scaffold; baseline (speedup 1.0000x reference)
import jax
import jax.numpy as jnp
from jax.experimental import pallas as pl


def kernel(x, edge_index, ids, W1, a1s, a1d, W2, a2s, a2d, Wc, acs, acd, fc_w, fc_b, sc_w, sc_b):
    raise NotImplementedError("write your pallas kernel here")


# speedup vs baseline: 8.4168x; 8.4168x over previous
import jax
import jax.numpy as jnp
from jax.experimental import pallas as pl

N = 10000
E = 320000
NFEAT = 128
NHID = 128
NHEAD = 8
DH = NHID // NHEAD
NTURNS = 4
ALPHA = 0.2

_ROW_BLK = 1000
_EDGE_BLK = 4000


def _node_kernel(h_ref, w_ref, asf_ref, adf_ref, s8_ref, z_ref, es_ref, ed_ref):
    hp = jax.lax.Precision.HIGHEST
    z = jnp.dot(h_ref[...], w_ref[...], precision=hp,
                preferred_element_type=jnp.float32)
    z_ref[...] = z
    s8 = s8_ref[...]
    es_ref[...] = jnp.dot(z * asf_ref[...], s8, precision=hp,
                          preferred_element_type=jnp.float32)
    ed_ref[...] = jnp.dot(z * adf_ref[...], s8, precision=hp,
                          preferred_element_type=jnp.float32)


def _edge_kernel(ess_ref, edd_ref, zs_ref, em_ref, r_ref, attn_ref, vals_ref):
    s = ess_ref[...] + edd_ref[...]
    s = jnp.where(s > 0, s, ALPHA * s)
    a8 = jnp.exp(s) * em_ref[...]
    attn_ref[...] = a8
    vals_ref[...] = zs_ref[...] * jnp.dot(
        a8, r_ref[...], precision=jax.lax.Precision.HIGHEST,
        preferred_element_type=jnp.float32)


def _comb_kernel(num_ref, den_ref, dm_ref, h_ref, r_ref, out_ref):
    denb = jnp.dot(den_ref[...], r_ref[...],
                   precision=jax.lax.Precision.HIGHEST,
                   preferred_element_type=jnp.float32) + 1e-9
    v = num_ref[...] / denb
    v = jnp.where(v > 0, v, jnp.exp(jnp.minimum(v, 0.0)) - 1.0)
    out_ref[...] = jnp.where(dm_ref[...] > 0, v, h_ref[...])


def _readout_kernel(h_ref, ids_ref, fcw_ref, fcb_ref, scw_ref, scb_ref,
                    s1_ref, s2_ref):
    h = h_ref[...]
    ids = ids_ref[...]

    def score_of(tt):
        m = (ids == tt).astype(jnp.float32)
        hm = jnp.sum(h * m, axis=0, keepdims=True)
        hh = hm / (jnp.sum(m) + 1e-9)
        hp = jax.lax.Precision.HIGHEST
        u = jnp.dot(hh, fcw_ref[...], precision=hp,
                    preferred_element_type=jnp.float32)
        u = jnp.maximum(u + fcb_ref[...], 0.0)
        return jnp.dot(u, scw_ref[...], precision=hp,
                       preferred_element_type=jnp.float32) + scb_ref[...]

    s1_ref[...] = score_of(NTURNS - 2)
    s2_ref[...] = score_of(NTURNS - 1)


def _node_call(h, W, asf, adf, s8):
    grid = (N // _ROW_BLK,)
    row = lambda i: (i, 0)
    rep = lambda i: (0, 0)
    return pl.pallas_call(
        _node_kernel,
        grid=grid,
        in_specs=[
            pl.BlockSpec((_ROW_BLK, NHID), row),
            pl.BlockSpec((NFEAT, NHID), rep),
            pl.BlockSpec((1, NHID), rep),
            pl.BlockSpec((1, NHID), rep),
            pl.BlockSpec((NHID, NHEAD), rep),
        ],
        out_specs=[
            pl.BlockSpec((_ROW_BLK, NHID), row),
            pl.BlockSpec((_ROW_BLK, NHEAD), row),
            pl.BlockSpec((_ROW_BLK, NHEAD), row),
        ],
        out_shape=[
            jax.ShapeDtypeStruct((N, NHID), jnp.float32),
            jax.ShapeDtypeStruct((N, NHEAD), jnp.float32),
            jax.ShapeDtypeStruct((N, NHEAD), jnp.float32),
        ],
    )(h, W, asf, adf, s8)


def _edge_call(ess, edd, zs, em, r):
    grid = (E // _EDGE_BLK,)
    row8 = lambda i: (i, 0)
    rep = lambda i: (0, 0)
    return pl.pallas_call(
        _edge_kernel,
        grid=grid,
        in_specs=[
            pl.BlockSpec((_EDGE_BLK, NHEAD), row8),
            pl.BlockSpec((_EDGE_BLK, NHEAD), row8),
            pl.BlockSpec((_EDGE_BLK, NHID), row8),
            pl.BlockSpec((_EDGE_BLK, 1), row8),
            pl.BlockSpec((NHEAD, NHID), rep),
        ],
        out_specs=[
            pl.BlockSpec((_EDGE_BLK, NHEAD), row8),
            pl.BlockSpec((_EDGE_BLK, NHID), row8),
        ],
        out_shape=[
            jax.ShapeDtypeStruct((E, NHEAD), jnp.float32),
            jax.ShapeDtypeStruct((E, NHID), jnp.float32),
        ],
    )(ess, edd, zs, em, r)


def _comb_call(num, den, dm, h, r):
    grid = (N // _ROW_BLK,)
    row = lambda i: (i, 0)
    rep = lambda i: (0, 0)
    return pl.pallas_call(
        _comb_kernel,
        grid=grid,
        in_specs=[
            pl.BlockSpec((_ROW_BLK, NHID), row),
            pl.BlockSpec((_ROW_BLK, NHEAD), row),
            pl.BlockSpec((_ROW_BLK, 1), row),
            pl.BlockSpec((_ROW_BLK, NHID), row),
            pl.BlockSpec((NHEAD, NHID), rep),
        ],
        out_specs=pl.BlockSpec((_ROW_BLK, NHID), row),
        out_shape=jax.ShapeDtypeStruct((N, NHID), jnp.float32),
    )(num, den, dm, h, r)


def _readout_call(h, ids2, fc_w, fc_b2, sc_w, sc_b2):
    rep = lambda: (0, 0)
    return pl.pallas_call(
        _readout_kernel,
        grid=(),
        in_specs=[
            pl.BlockSpec((N, NHID), rep),
            pl.BlockSpec((N, 1), rep),
            pl.BlockSpec((NHID, 2 * NHID), rep),
            pl.BlockSpec((1, 2 * NHID), rep),
            pl.BlockSpec((2 * NHID, 1), rep),
            pl.BlockSpec((1, 1), rep),
        ],
        out_specs=[
            pl.BlockSpec((1, 1), rep),
            pl.BlockSpec((1, 1), rep),
        ],
        out_shape=[
            jax.ShapeDtypeStruct((1, 1), jnp.float32),
            jax.ShapeDtypeStruct((1, 1), jnp.float32),
        ],
    )(h, ids2, fc_w, fc_b2, sc_w, sc_b2)


def kernel(x, edge_index, ids, W1, a1s, a1d, W2, a2s, a2d, Wc, acs, acd,
           fc_w, fc_b, sc_w, sc_b):
    src = edge_index[0]
    dst = edge_index[1]
    its = jnp.take(ids, src)
    itd = jnp.take(ids, dst)

    lane_head = jnp.arange(NHID, dtype=jnp.int32) // DH
    r = (lane_head[None, :] == jnp.arange(NHEAD, dtype=jnp.int32)[:, None]
         ).astype(jnp.float32)
    s8 = r.T

    def gat(h, W, a_s, a_d, em, dm):
        z, es, ed = _node_call(h, W, a_s.reshape(1, NHID),
                               a_d.reshape(1, NHID), s8)
        ess = jnp.take(es, src, axis=0)
        edd = jnp.take(ed, dst, axis=0)
        zs = jnp.take(z, src, axis=0)
        attn8, vals = _edge_call(ess, edd, zs, em.reshape(E, 1), r)
        num = jax.ops.segment_sum(vals, dst, num_segments=N)
        den = jax.ops.segment_sum(attn8, dst, num_segments=N)
        return _comb_call(num, den, dm.reshape(N, 1), h, r)

    h = x
    for t in range(NTURNS):
        dm = (ids == t).astype(jnp.float32)
        em = ((its == t) & (itd == t)).astype(jnp.float32)
        if t % 2 == 0:
            h = gat(h, W1, a1s, a1d, em, dm)
        else:
            h = gat(h, W2, a2s, a2d, em, dm)
        if t > 0:
            em2 = ((its == t - 1) & (itd == t)).astype(jnp.float32)
            h = gat(h, Wc, acs, acd, em2, dm)

    s1, s2 = _readout_call(h, ids.reshape(N, 1), fc_w,
                           fc_b.reshape(1, 2 * NHID), sc_w,
                           sc_b.reshape(1, 1))
    return (s1.reshape(()), s2.reshape(()))
